# Initial kernel scaffold; baseline (speedup 1.0000x reference)
import jax
import jax.numpy as jnp
from jax.experimental import pallas as pl


def kernel(x, experts, W):
    raise NotImplementedError("write your pallas kernel here")


# speedup vs baseline: 1.2803x; 1.2803x over previous
import functools

import jax
import jax.numpy as jnp
from jax.experimental import pallas as pl
from jax.experimental.pallas import tpu as pltpu


def _gate_body(x_ref, w_ref, e_ref, o_ref):
    xb = x_ref[...]
    wt = w_ref[...]
    logits = jax.lax.dot_general(
        xb, wt, (((1,), (1,)), ((), ())),
        preferred_element_type=jnp.float32)
    m = jnp.max(logits, axis=1, keepdims=True)
    p = jnp.exp(logits - m)
    p = p / jnp.sum(p, axis=1, keepdims=True)
    num = e_ref.shape[1]
    acc = p[:, 0:1] * e_ref[:, 0, :]
    for n in range(1, num):
        acc = acc + p[:, n:n + 1] * e_ref[:, n, :]
    o_ref[...] = acc


@jax.jit
def kernel(x, experts, W):
    B, D = x.shape
    NUM = W.shape[0]
    BLK = 256
    grid = (B // BLK,)
    return pl.pallas_call(
        _gate_body,
        grid=grid,
        in_specs=[
            pl.BlockSpec((BLK, D), lambda i: (i, 0)),
            pl.BlockSpec((NUM, D), lambda i: (0, 0)),
            pl.BlockSpec((BLK, NUM, D), lambda i: (i, 0, 0)),
        ],
        out_specs=pl.BlockSpec((BLK, D), lambda i: (i, 0)),
        out_shape=jax.ShapeDtypeStruct((B, D), jnp.float32),
    )(x, W, experts)
